# Initial kernel scaffold; baseline (speedup 1.0000x reference)
import jax
import jax.numpy as jnp
from jax.experimental import pallas as pl


def kernel(x, edge_index, params):
    raise NotImplementedError("write your pallas kernel here")


# speedup vs baseline: 11.1811x; 11.1811x over previous
import numpy as np
import jax
import jax.numpy as jnp
from jax import lax
from jax.experimental import pallas as pl
from jax.experimental.pallas import tpu as pltpu
from jax.experimental.pallas import tpu_sc as plsc

_NC = 2
_NS = 16
_NW = _NC * _NS
_CH = 128
_L = 16


def _sc_spmm(h, row3, col3, n_pad):
    N, D = h.shape
    _, J, _ = row3.shape
    rows_per_tile = n_pad // _NS

    def body(h_hbm, row_hbm, col_hbm, out_hbm, row_v, col_v, buf, acc, sem):
        c = lax.axis_index("c")
        s = lax.axis_index("s")
        g = c * _NS + s
        zero16 = jnp.zeros((_L,), jnp.float32)

        def _zb(r, _):
            for kk in range(D // _L):
                buf[r, pl.ds(kk * _L, _L)] = zero16
            return 0
        lax.fori_loop(0, _CH, _zb, 0)

        base = s * rows_per_tile
        nfull = rows_per_tile // _CH
        rem = rows_per_tile - nfull * _CH
        for kk in range(nfull):
            pltpu.sync_copy(buf, acc.at[pl.ds(base + kk * _CH, _CH)])
        if rem:
            pltpu.sync_copy(buf.at[pl.ds(0, rem)],
                            acc.at[pl.ds(base + nfull * _CH, rem)])

        pltpu.sync_copy(row_hbm.at[g], row_v)
        pltpu.sync_copy(col_hbm.at[g], col_v)

        plsc.subcore_barrier()

        def _ej(j, _):
            pltpu.async_copy(h_hbm.at[col_v.at[j]], buf, sem).wait()
            pltpu.sync_copy(buf, acc.at[row_v.at[j]], add=True)
            return 0
        lax.fori_loop(0, J, _ej, 0)

        plsc.subcore_barrier()

        pltpu.sync_copy(acc.at[pl.ds(base, rows_per_tile)],
                        out_hbm.at[c, pl.ds(base, rows_per_tile)])

    mesh = plsc.VectorSubcoreMesh(core_axis_name="c", subcore_axis_name="s")
    kfn = pl.kernel(
        body,
        out_type=jax.ShapeDtypeStruct((_NC, n_pad, D), jnp.float32),
        mesh=mesh,
        scratch_types=[
            pltpu.VMEM((J, _CH), jnp.int32),
            pltpu.VMEM((J, _CH), jnp.int32),
            pltpu.VMEM((_CH, D), jnp.float32),
            pltpu.VMEM_SHARED((n_pad, D), jnp.float32),
            pltpu.SemaphoreType.DMA,
        ],
        compiler_params=pltpu.CompilerParams(needs_layout_passes=False))
    return kfn(h, row3, col3)


def _sc_deg(row3, hist_r):
    _, J, _ = row3.shape
    hist_per_tile = hist_r // _NS

    def body(row_hbm, deg_hbm, row_v, hist1, hist, idx_h, hist_sh, sem):
        c = lax.axis_index("c")
        s = lax.axis_index("s")
        g = c * _NS + s
        zero16 = jnp.zeros((_L,), jnp.float32)

        def _zh(i, _):
            hist1[pl.ds(i * _L, _L)] = zero16
            return 0
        lax.fori_loop(0, hist_r * 128 // _L, _zh, 0)

        def _zh2(r, _):
            for kk in range(128 // _L):
                hist[r, pl.ds(kk * _L, _L)] = zero16
            return 0
        lax.fori_loop(0, hist_per_tile, _zh2, 0)
        pltpu.sync_copy(hist.at[pl.ds(0, hist_per_tile)],
                        hist_sh.at[pl.ds(s * hist_per_tile, hist_per_tile)])

        iota16 = lax.iota(jnp.int32, _L)
        for q in range(hist_r // _L):
            idx_h[0, pl.ds(q * _L, _L)] = iota16 + q * _L

        pltpu.sync_copy(row_hbm.at[g], row_v)

        ones16 = jnp.ones((_L,), jnp.float32)

        def _dh(j, _):
            for kk in range(_CH // _L):
                v = row_v[j, pl.ds(kk * _L, _L)]
                plsc.addupdate_scatter(hist1, [v], ones16)
            return 0
        lax.fori_loop(0, J, _dh, 0)

        def _cp(r, _):
            for kk in range(128 // _L):
                hist[r, pl.ds(kk * _L, _L)] = hist1[pl.ds(r * 128 + kk * _L, _L)]
            return 0
        lax.fori_loop(0, hist_r, _cp, 0)

        plsc.subcore_barrier()
        pltpu.sync_copy(hist, hist_sh.at[idx_h.at[0]], add=True)
        plsc.subcore_barrier()

        pltpu.sync_copy(hist_sh.at[pl.ds(s * hist_per_tile, hist_per_tile)],
                        deg_hbm.at[c, pl.ds(s * hist_per_tile, hist_per_tile)])

    mesh = plsc.VectorSubcoreMesh(core_axis_name="c", subcore_axis_name="s")
    kfn = pl.kernel(
        body,
        out_type=jax.ShapeDtypeStruct((_NC, hist_r, 128), jnp.float32),
        mesh=mesh,
        scratch_types=[
            pltpu.VMEM((J, _CH), jnp.int32),
            pltpu.VMEM((hist_r * 128,), jnp.float32),
            pltpu.VMEM((hist_r, 128), jnp.float32),
            pltpu.VMEM((1, hist_r), jnp.int32),
            pltpu.VMEM_SHARED((hist_r, 128), jnp.float32),
            pltpu.SemaphoreType.DMA,
        ],
        compiler_params=pltpu.CompilerParams(needs_layout_passes=False))
    return kfn(row3)


_BLK = 1000


def _ln_prelu(z, g, b, a):
    mu = jnp.mean(z, axis=-1, keepdims=True)
    var = jnp.mean((z - mu) ** 2, axis=-1, keepdims=True)
    zn = (z - mu) * lax.rsqrt(var + 1e-5) * g + b
    return jnp.where(zn >= 0, zn, a * zn)


def _tc_layer1(h, n0, n1, d0, d1, wsT, bs, wnT, g, b, a):
    N, D = h.shape

    def body(h_ref, n0_ref, n1_ref, d0_ref, d1_ref,
             ws_ref, bs_ref, wn_ref, g_ref, b_ref, a_ref,
             ho_ref, inv_ref):
        inv = 1.0 / jnp.maximum(d0_ref[...] + d1_ref[...], 1.0)
        neigh = (n0_ref[...] + n1_ref[...]) * inv
        hh = h_ref[...]
        z = (jnp.dot(hh, ws_ref[...], preferred_element_type=jnp.float32)
             + jnp.dot(neigh, wn_ref[...], preferred_element_type=jnp.float32)
             + bs_ref[...])
        zp = _ln_prelu(z, g_ref[...], b_ref[...], a_ref[0, 0])
        ho_ref[...] = zp + hh
        inv_ref[...] = inv

    grid = (N // _BLK,)
    mat = pl.BlockSpec((_BLK, D), lambda i: (i, 0))
    colv = pl.BlockSpec((_BLK, 1), lambda i: (i, 0))
    wfull = pl.BlockSpec((D, D), lambda i: (0, 0))
    rowv = pl.BlockSpec((1, D), lambda i: (0, 0))
    scal = pl.BlockSpec((1, 1), lambda i: (0, 0))
    return pl.pallas_call(
        body,
        grid=grid,
        in_specs=[mat, mat, mat, colv, colv, wfull, rowv, wfull, rowv, rowv, scal],
        out_specs=[mat, colv],
        out_shape=[jax.ShapeDtypeStruct((N, D), jnp.float32),
                   jax.ShapeDtypeStruct((N, 1), jnp.float32)],
    )(h, n0, n1, d0, d1, wsT, bs, wnT, g, b, a)


def _tc_layer2_head(h, n0, n1, inv, wsT, bs, wnT, g, b, a,
                    w1T, b1, g2, b2, a2, w2T, b2b):
    N, D = h.shape

    def body(h_ref, n0_ref, n1_ref, inv_ref,
             ws_ref, bs_ref, wn_ref, g_ref, b_ref, a_ref,
             w1_ref, b1_ref, g2_ref, b2_ref, a2_ref, w2_ref, b2b_ref,
             out_ref):
        neigh = (n0_ref[...] + n1_ref[...]) * inv_ref[...]
        hh = h_ref[...]
        z = (jnp.dot(hh, ws_ref[...], preferred_element_type=jnp.float32)
             + jnp.dot(neigh, wn_ref[...], preferred_element_type=jnp.float32)
             + bs_ref[...])
        h2 = _ln_prelu(z, g_ref[...], b_ref[...], a_ref[0, 0]) + hh
        z2 = jnp.dot(h2, w1_ref[...], preferred_element_type=jnp.float32) + b1_ref[...]
        z2 = _ln_prelu(z2, g2_ref[...], b2_ref[...], a2_ref[0, 0])
        out_ref[...] = (jnp.sum(z2 * w2_ref[...], axis=-1, keepdims=True)
                        + b2b_ref[0, 0])

    grid = (N // _BLK,)
    mat = pl.BlockSpec((_BLK, D), lambda i: (i, 0))
    colv = pl.BlockSpec((_BLK, 1), lambda i: (i, 0))
    wfull = pl.BlockSpec((D, D), lambda i: (0, 0))
    rowv = pl.BlockSpec((1, D), lambda i: (0, 0))
    scal = pl.BlockSpec((1, 1), lambda i: (0, 0))
    return pl.pallas_call(
        body,
        grid=grid,
        in_specs=[mat, mat, mat, colv,
                  wfull, rowv, wfull, rowv, rowv, scal,
                  wfull, rowv, rowv, rowv, scal, rowv, scal],
        out_specs=colv,
        out_shape=jax.ShapeDtypeStruct((N, 1), jnp.float32),
    )(h, n0, n1, inv, wsT, bs, wnT, g, b, a,
      w1T, b1, g2, b2, a2, w2T, b2b)


def kernel(x, edge_index, params):
    N, D = x.shape
    E = edge_index.shape[1]
    J = -(-E // (_NW * _CH))
    E_pad = _NW * J * _CH
    rows_per_tile = -(-(N + _L) // _NS)
    rows_per_tile = -(-rows_per_tile // 8) * 8
    n_pad = rows_per_tile * _NS
    hist_rows = -(-n_pad // 128)
    hist_per_tile = -(-hist_rows // _NS)
    hist_per_tile = -(-hist_per_tile // 8) * 8
    hist_r = hist_per_tile * _NS

    row = edge_index[0]
    col = edge_index[1]
    pad = E_pad - E
    if pad:
        pr = jnp.asarray(np.arange(pad) % _L + N, jnp.int32)
        pc = jnp.asarray(np.arange(pad) % N, jnp.int32)
        row = jnp.concatenate([row, pr])
        col = jnp.concatenate([col, pc])
    row3 = row.reshape(_NW, J, _CH)
    col3 = col.reshape(_NW, J, _CH)

    blocks = params["blocks"]
    head = params["head"]

    dparts = _sc_deg(row3, hist_r)
    nparts = _sc_spmm(x, row3, col3, n_pad)
    dflat = dparts.reshape(_NC, hist_r * 128)[:, :N]
    b0 = blocks[0]
    h1, inv = _tc_layer1(
        x, nparts[0, :N], nparts[1, :N],
        dflat[0].reshape(N, 1), dflat[1].reshape(N, 1),
        b0["Wself"].T, b0["bself"].reshape(1, D), b0["Wneigh"].T,
        b0["ln_g"].reshape(1, D), b0["ln_b"].reshape(1, D),
        b0["a"].reshape(1, 1))

    nparts2 = _sc_spmm(h1, row3, col3, n_pad)
    b1 = blocks[1]
    out = _tc_layer2_head(
        h1, nparts2[0, :N], nparts2[1, :N], inv,
        b1["Wself"].T, b1["bself"].reshape(1, D), b1["Wneigh"].T,
        b1["ln_g"].reshape(1, D), b1["ln_b"].reshape(1, D),
        b1["a"].reshape(1, 1),
        head["W1"].T, head["b1"].reshape(1, D),
        head["ln_g"].reshape(1, D), head["ln_b"].reshape(1, D),
        head["a"].reshape(1, 1),
        head["W2"].reshape(1, D), head["b2"].reshape(1, 1))
    return out[:, 0]
